# baseline (speedup 1.0000x reference)
import functools

import jax
import jax.numpy as jnp
from jax import lax
from jax.experimental import pallas as pl
from jax.experimental.pallas import tpu as pltpu
from jax.experimental.pallas import tpu_sc as plsc

NN = 10000
NPAD = 10112
DHID = 128
DH = 64
DOUT = 40
DOUT_PAD = 64
NCORE = 2
NSUB = 16
NW = NCORE * NSUB
BK = 128
NOP0 = 160
NOP1 = 0
TOTBLK = NSUB * (NOP0 + NOP1)
TOTBLK_PAD = TOTBLK + NOP0
EPAD = TOTBLK_PAD * BK
ROWS_SUB = NPAD // NSUB
NRING = 4
SLAG = 1


def _core_base_cnt(c, s):
    base = jnp.where(c == 0, s * NOP0, NSUB * NOP0 + s * NOP1)
    cnt = jnp.where(c == 0, NOP0, NOP1)
    return base, cnt


def _vmesh():
    return plsc.VectorSubcoreMesh(core_axis_name="c", subcore_axis_name="s")


def _sc_degree(dst_idx, ones16, zeros16):
    @functools.partial(
        pl.kernel,
        out_type=jax.ShapeDtypeStruct((NCORE, NPAD, 16), jnp.float32),
        mesh=_vmesh(),
        compiler_params=pltpu.CompilerParams(use_tc_tiling_on_sc=False),
        scratch_types=[
            pltpu.VMEM((NOP0, BK), jnp.int32),
            pltpu.VMEM((BK, 16), jnp.float32),
            pltpu.VMEM_SHARED((NPAD, 16), jnp.float32),
        ],
    )
    def k(dst_hbm, ones_hbm, zeros_hbm, out_hbm, dst_v, ones_v, acc):
        c = lax.axis_index("c")
        s = lax.axis_index("s")
        base, cnt = _core_base_cnt(c, s)
        pltpu.sync_copy(zeros_hbm, acc.at[pl.ds(s * ROWS_SUB, ROWS_SUB)])
        pltpu.sync_copy(dst_hbm.at[pl.ds(base, NOP0)], dst_v)
        pltpu.sync_copy(ones_hbm, ones_v)
        plsc.subcore_barrier()

        @pl.loop(0, cnt)
        def _(i):
            pltpu.sync_copy(ones_v, acc.at[dst_v.at[i]], add=True)

        plsc.subcore_barrier()
        pltpu.sync_copy(acc.at[pl.ds(s * ROWS_SUB, ROWS_SUB)],
                        out_hbm.at[c, pl.ds(s * ROWS_SUB, ROWS_SUB)])

    return k(dst_idx, ones16, zeros16)


def _sc_aggregate(h, src_idx, dst_idx, zeros):
    @functools.partial(
        pl.kernel,
        out_type=jax.ShapeDtypeStruct((NCORE, NPAD, DH), jnp.float32),
        mesh=_vmesh(),
        compiler_params=pltpu.CompilerParams(use_tc_tiling_on_sc=False),
        scratch_types=[
            pltpu.VMEM((NOP0, BK), jnp.int32),
            pltpu.VMEM((NOP0, BK), jnp.int32),
            pltpu.VMEM((NRING, BK, DH), jnp.float32),
            pltpu.VMEM_SHARED((NPAD, DH), jnp.float32),
            pltpu.SemaphoreType.DMA((NRING,)),
            pltpu.SemaphoreType.DMA((NRING,)),
        ],
    )
    def k(h_hbm, src_hbm, dst_hbm, zeros_hbm, out_hbm,
          src_v, dst_v, rows_v, acc, gsem, ssem):
        c = lax.axis_index("c")
        s = lax.axis_index("s")
        base, cnt = _core_base_cnt(c, s)
        pltpu.sync_copy(zeros_hbm, acc.at[pl.ds(s * ROWS_SUB, ROWS_SUB)])
        pltpu.sync_copy(src_hbm.at[pl.ds(base, NOP0)], src_v)
        pltpu.sync_copy(dst_hbm.at[pl.ds(base, NOP0)], dst_v)
        plsc.subcore_barrier()

        for b in range(NRING):
            @pl.when(cnt > 0)
            def _():
                pltpu.async_copy(h_hbm.at[src_v.at[b]], rows_v.at[b],
                                 gsem.at[b])

        @pl.loop(0, cnt, step=NRING)
        def _(g):
            for b in range(NRING):
                blk = g + b
                pltpu.make_async_copy(h_hbm.at[src_v.at[blk]], rows_v.at[b],
                                      gsem.at[b]).wait()
                pltpu.async_copy(rows_v.at[b], acc.at[dst_v.at[blk]],
                                 ssem.at[b], add=True)
                j = blk - SLAG
                bj = (b - SLAG) % NRING

                @pl.when(jnp.logical_and(j >= 0, j + NRING < cnt))
                def _():
                    pltpu.make_async_copy(rows_v.at[bj], acc.at[dst_v.at[j]],
                                          ssem.at[bj]).wait()
                    pltpu.async_copy(h_hbm.at[src_v.at[j + NRING]],
                                     rows_v.at[bj], gsem.at[bj])

        for i in range(NRING):
            j = cnt - NRING + i

            @pl.when(cnt > 0)
            def _():
                pltpu.make_async_copy(rows_v.at[i], acc.at[dst_v.at[j]],
                                      ssem.at[i]).wait()

        plsc.subcore_barrier()
        pltpu.sync_copy(acc.at[pl.ds(s * ROWS_SUB, ROWS_SUB)],
                        out_hbm.at[c, pl.ds(s * ROWS_SUB, ROWS_SUB)])

    return k(h, src_idx, dst_idx, zeros)


def _dinv_from(deg_ref):
    deg = deg_ref[0, :, 0:1] + deg_ref[1, :, 0:1] + 1.0
    return lax.rsqrt(deg)


def _tc_first(xp, W, degp):
    def body(x_ref, w_ref, deg_ref, oa_ref, ob_ref):
        dinv = _dinv_from(deg_ref)
        h = jnp.dot(x_ref[...], w_ref[...], preferred_element_type=jnp.float32)
        h = h * dinv
        oa_ref[...] = h[:, :DH]
        ob_ref[...] = h[:, DH:]

    return pl.pallas_call(
        body,
        out_shape=[jax.ShapeDtypeStruct((NPAD, DH), jnp.float32),
                   jax.ShapeDtypeStruct((NPAD, DH), jnp.float32)],
    )(xp, W, degp)


def _tc_mid(pa, pb, ha, hb, degp, b, g, be, W, two_halves):
    def body(pa_ref, pb_ref, ha_ref, hb_ref, deg_ref, b_ref, g_ref, be_ref,
             w_ref, *o_refs):
        dinv = _dinv_from(deg_ref)
        za = (pa_ref[0] + pa_ref[1] + ha_ref[...]) * dinv + b_ref[:, :DH]
        zb = (pb_ref[0] + pb_ref[1] + hb_ref[...]) * dinv + b_ref[:, DH:]
        z = jnp.concatenate([za, zb], axis=1)
        rows = lax.broadcasted_iota(jnp.int32, (NPAD, 1), 0)
        mask = rows < NN
        zm = jnp.where(mask, z, 0.0)
        mu = jnp.sum(zm, axis=0, keepdims=True) * (1.0 / NN)
        zc = jnp.where(mask, z - mu, 0.0)
        var = jnp.sum(zc * zc, axis=0, keepdims=True) * (1.0 / NN)
        zn = zc * lax.rsqrt(var + 1e-5) * g_ref[...] + be_ref[...]
        act = jnp.where(mask, jnp.maximum(zn, 0.0), 0.0)
        h = jnp.dot(act, w_ref[...], preferred_element_type=jnp.float32)
        h = h * dinv
        if two_halves:
            o_refs[0][...] = h[:, :DH]
            o_refs[1][...] = h[:, DH:]
        else:
            o_refs[0][...] = h

    n_out = 2 if two_halves else 1
    return pl.pallas_call(
        body,
        out_shape=[jax.ShapeDtypeStruct((NPAD, DH), jnp.float32)] * n_out,
        compiler_params=pltpu.CompilerParams(vmem_limit_bytes=64 * 1024 * 1024),
    )(pa, pb, ha, hb, degp, b, g, be, W)


def _tc_final(p, hp, degp, b):
    def body(p_ref, h_ref, deg_ref, b_ref, o_ref):
        dinv = _dinv_from(deg_ref)
        z = (p_ref[0] + p_ref[1] + h_ref[...]) * dinv + b_ref[...]
        cols = lax.broadcasted_iota(jnp.int32, (1, DOUT_PAD), 1)
        cmask = cols < DOUT
        zneg = jnp.where(cmask, z, -jnp.inf)
        m = jnp.max(zneg, axis=1, keepdims=True)
        e = jnp.where(cmask, jnp.exp(z - m), 0.0)
        lse = jnp.log(jnp.sum(e, axis=1, keepdims=True))
        ls = z - m - lse
        o_ref[...] = ls[:NN, :DOUT]

    return pl.pallas_call(
        body,
        out_shape=jax.ShapeDtypeStruct((NN, DOUT), jnp.float32),
    )(p, hp, degp, b)


def kernel(x, edge_index, W1, b1, g1, be1, W2, b2, g2, be2, W3, b3):
    src = edge_index[0]
    dst = edge_index[1]
    npad_e = EPAD - src.shape[0]
    fill = jnp.full((npad_e,), NN, dtype=src.dtype)
    src_r = jnp.concatenate([src, fill]).reshape(TOTBLK_PAD, BK)
    dst_r = jnp.concatenate([dst, fill]).reshape(TOTBLK_PAD, BK)
    dst_r2 = dst_r

    ones16 = jnp.ones((BK, 16), jnp.float32)
    zeros16 = jnp.zeros((ROWS_SUB, 16), jnp.float32)
    zeros64 = jnp.zeros((ROWS_SUB, DH), jnp.float32)

    xp = jnp.pad(x, ((0, NPAD - NN), (0, 0)))
    W3p = jnp.pad(W3, ((0, 0), (0, DOUT_PAD - DOUT)))
    b1r, g1r, be1r = b1.reshape(1, -1), g1.reshape(1, -1), be1.reshape(1, -1)
    b2r, g2r, be2r = b2.reshape(1, -1), g2.reshape(1, -1), be2.reshape(1, -1)
    b3r = jnp.pad(b3, (0, DOUT_PAD - DOUT)).reshape(1, -1)

    degp = _sc_degree(dst_r2, ones16, zeros16)
    h1a, h1b = _tc_first(xp, W1, degp)
    p1a = _sc_aggregate(h1a, src_r, dst_r, zeros64)
    p1b = _sc_aggregate(h1b, src_r, dst_r, zeros64)
    h2a, h2b = _tc_mid(p1a, p1b, h1a, h1b, degp, b1r, g1r, be1r, W2, True)
    p2a = _sc_aggregate(h2a, src_r, dst_r, zeros64)
    p2b = _sc_aggregate(h2b, src_r, dst_r, zeros64)
    (h3,) = _tc_mid(p2a, p2b, h2a, h2b, degp, b2r, g2r, be2r, W3p, False)
    p3 = _sc_aggregate(h3, src_r, dst_r, zeros64)
    return _tc_final(p3, h3, degp, b3r)


# speedup vs baseline: 3.3749x; 3.3749x over previous
import functools

import jax
import jax.numpy as jnp
from jax import lax
from jax.experimental import pallas as pl
from jax.experimental.pallas import tpu as pltpu
from jax.experimental.pallas import tpu_sc as plsc

NN = 10000
NPAD = 10112
DHID = 128
DH = 64
DOUT = 40
DOUT_PAD = 64
NCORE = 2
NSUB = 16
NW = NCORE * NSUB
BK = 128
NOP0 = 80
NOP1 = 80
TOTBLK = NSUB * (NOP0 + NOP1)
TOTBLK_PAD = TOTBLK + NOP0
EPAD = TOTBLK_PAD * BK
ROWS_SUB = NPAD // NSUB
NRING = 4
SLAG = 1


def _core_base_cnt(c, s):
    base = jnp.where(c == 0, s * NOP0, NSUB * NOP0 + s * NOP1)
    cnt = jnp.where(c == 0, NOP0, NOP1)
    return base, cnt


def _vmesh():
    return plsc.VectorSubcoreMesh(core_axis_name="c", subcore_axis_name="s")


def _sc_degree(dst_idx, ones16, zeros16):
    @functools.partial(
        pl.kernel,
        out_type=jax.ShapeDtypeStruct((NCORE, NPAD, 16), jnp.float32),
        mesh=_vmesh(),
        compiler_params=pltpu.CompilerParams(use_tc_tiling_on_sc=False),
        scratch_types=[
            pltpu.VMEM((NOP0, BK), jnp.int32),
            pltpu.VMEM((BK, 16), jnp.float32),
            pltpu.VMEM_SHARED((NPAD, 16), jnp.float32),
        ],
    )
    def k(dst_hbm, ones_hbm, zeros_hbm, out_hbm, dst_v, ones_v, acc):
        c = lax.axis_index("c")
        s = lax.axis_index("s")
        base, cnt = _core_base_cnt(c, s)
        pltpu.sync_copy(zeros_hbm, acc.at[pl.ds(s * ROWS_SUB, ROWS_SUB)])
        pltpu.sync_copy(dst_hbm.at[pl.ds(base, NOP0)], dst_v)
        pltpu.sync_copy(ones_hbm, ones_v)
        plsc.subcore_barrier()

        @pl.loop(0, cnt)
        def _(i):
            pltpu.sync_copy(ones_v, acc.at[dst_v.at[i]], add=True)

        plsc.subcore_barrier()
        pltpu.sync_copy(acc.at[pl.ds(s * ROWS_SUB, ROWS_SUB)],
                        out_hbm.at[c, pl.ds(s * ROWS_SUB, ROWS_SUB)])

    return k(dst_idx, ones16, zeros16)


def _sc_aggregate(h, src_idx, dst_idx, zeros):
    @functools.partial(
        pl.kernel,
        out_type=jax.ShapeDtypeStruct((NCORE, NPAD, DH), jnp.float32),
        mesh=_vmesh(),
        compiler_params=pltpu.CompilerParams(use_tc_tiling_on_sc=False),
        scratch_types=[
            pltpu.VMEM((NOP0, BK), jnp.int32),
            pltpu.VMEM((NOP0, BK), jnp.int32),
            pltpu.VMEM((NRING, BK, DH), jnp.float32),
            pltpu.VMEM_SHARED((NPAD, DH), jnp.float32),
            pltpu.SemaphoreType.DMA((NRING,)),
            pltpu.SemaphoreType.DMA((NRING,)),
        ],
    )
    def k(h_hbm, src_hbm, dst_hbm, zeros_hbm, out_hbm,
          src_v, dst_v, rows_v, acc, gsem, ssem):
        c = lax.axis_index("c")
        s = lax.axis_index("s")
        base, cnt = _core_base_cnt(c, s)
        pltpu.sync_copy(zeros_hbm, acc.at[pl.ds(s * ROWS_SUB, ROWS_SUB)])
        pltpu.sync_copy(src_hbm.at[pl.ds(base, NOP0)], src_v)
        pltpu.sync_copy(dst_hbm.at[pl.ds(base, NOP0)], dst_v)
        plsc.subcore_barrier()

        for b in range(NRING):
            @pl.when(cnt > 0)
            def _():
                pltpu.async_copy(h_hbm.at[src_v.at[b]], rows_v.at[b],
                                 gsem.at[b])

        @pl.loop(0, cnt, step=NRING)
        def _(g):
            for b in range(NRING):
                blk = g + b
                pltpu.make_async_copy(h_hbm.at[src_v.at[blk]], rows_v.at[b],
                                      gsem.at[b]).wait()
                pltpu.async_copy(rows_v.at[b], acc.at[dst_v.at[blk]],
                                 ssem.at[b], add=True)
                j = blk - SLAG
                bj = (b - SLAG) % NRING

                @pl.when(jnp.logical_and(j >= 0, j + NRING < cnt))
                def _():
                    pltpu.make_async_copy(rows_v.at[bj], acc.at[dst_v.at[j]],
                                          ssem.at[bj]).wait()
                    pltpu.async_copy(h_hbm.at[src_v.at[j + NRING]],
                                     rows_v.at[bj], gsem.at[bj])

        for i in range(NRING):
            j = cnt - NRING + i

            @pl.when(cnt > 0)
            def _():
                pltpu.make_async_copy(rows_v.at[i], acc.at[dst_v.at[j]],
                                      ssem.at[i]).wait()

        plsc.subcore_barrier()
        pltpu.sync_copy(acc.at[pl.ds(s * ROWS_SUB, ROWS_SUB)],
                        out_hbm.at[c, pl.ds(s * ROWS_SUB, ROWS_SUB)])

    return k(h, src_idx, dst_idx, zeros)


def _dinv_from(deg_ref):
    deg = deg_ref[0, :, 0:1] + deg_ref[1, :, 0:1] + 1.0
    return lax.rsqrt(deg)


def _tc_first(xp, W, degp):
    def body(x_ref, w_ref, deg_ref, oa_ref, ob_ref):
        dinv = _dinv_from(deg_ref)
        h = jnp.dot(x_ref[...], w_ref[...], preferred_element_type=jnp.float32)
        h = h * dinv
        oa_ref[...] = h[:, :DH]
        ob_ref[...] = h[:, DH:]

    return pl.pallas_call(
        body,
        out_shape=[jax.ShapeDtypeStruct((NPAD, DH), jnp.float32),
                   jax.ShapeDtypeStruct((NPAD, DH), jnp.float32)],
    )(xp, W, degp)


def _tc_mid(pa, pb, ha, hb, degp, b, g, be, W, two_halves):
    def body(pa_ref, pb_ref, ha_ref, hb_ref, deg_ref, b_ref, g_ref, be_ref,
             w_ref, *o_refs):
        dinv = _dinv_from(deg_ref)
        za = (pa_ref[0] + pa_ref[1] + ha_ref[...]) * dinv + b_ref[:, :DH]
        zb = (pb_ref[0] + pb_ref[1] + hb_ref[...]) * dinv + b_ref[:, DH:]
        z = jnp.concatenate([za, zb], axis=1)
        rows = lax.broadcasted_iota(jnp.int32, (NPAD, 1), 0)
        mask = rows < NN
        zm = jnp.where(mask, z, 0.0)
        mu = jnp.sum(zm, axis=0, keepdims=True) * (1.0 / NN)
        zc = jnp.where(mask, z - mu, 0.0)
        var = jnp.sum(zc * zc, axis=0, keepdims=True) * (1.0 / NN)
        zn = zc * lax.rsqrt(var + 1e-5) * g_ref[...] + be_ref[...]
        act = jnp.where(mask, jnp.maximum(zn, 0.0), 0.0)
        h = jnp.dot(act, w_ref[...], preferred_element_type=jnp.float32)
        h = h * dinv
        if two_halves:
            o_refs[0][...] = h[:, :DH]
            o_refs[1][...] = h[:, DH:]
        else:
            o_refs[0][...] = h

    n_out = 2 if two_halves else 1
    return pl.pallas_call(
        body,
        out_shape=[jax.ShapeDtypeStruct((NPAD, DH), jnp.float32)] * n_out,
        compiler_params=pltpu.CompilerParams(vmem_limit_bytes=64 * 1024 * 1024),
    )(pa, pb, ha, hb, degp, b, g, be, W)


def _tc_final(p, hp, degp, b):
    def body(p_ref, h_ref, deg_ref, b_ref, o_ref):
        dinv = _dinv_from(deg_ref)
        z = (p_ref[0] + p_ref[1] + h_ref[...]) * dinv + b_ref[...]
        cols = lax.broadcasted_iota(jnp.int32, (1, DOUT_PAD), 1)
        cmask = cols < DOUT
        zneg = jnp.where(cmask, z, -jnp.inf)
        m = jnp.max(zneg, axis=1, keepdims=True)
        e = jnp.where(cmask, jnp.exp(z - m), 0.0)
        lse = jnp.log(jnp.sum(e, axis=1, keepdims=True))
        ls = z - m - lse
        o_ref[...] = ls[:NN, :DOUT]

    return pl.pallas_call(
        body,
        out_shape=jax.ShapeDtypeStruct((NN, DOUT), jnp.float32),
    )(p, hp, degp, b)


def kernel(x, edge_index, W1, b1, g1, be1, W2, b2, g2, be2, W3, b3):
    src = edge_index[0]
    dst = edge_index[1]
    npad_e = EPAD - src.shape[0]
    fill = NN + (jnp.arange(npad_e, dtype=src.dtype) % (NPAD - NN))
    src_r = jnp.concatenate([src, fill]).reshape(TOTBLK_PAD, BK)
    dst_r = jnp.concatenate([dst, fill]).reshape(TOTBLK_PAD, BK)
    dst_r2 = dst_r

    ones16 = jnp.ones((BK, 16), jnp.float32)
    zeros16 = jnp.zeros((ROWS_SUB, 16), jnp.float32)
    zeros64 = jnp.zeros((ROWS_SUB, DH), jnp.float32)

    xp = jnp.pad(x, ((0, NPAD - NN), (0, 0)))
    W3p = jnp.pad(W3, ((0, 0), (0, DOUT_PAD - DOUT)))
    b1r, g1r, be1r = b1.reshape(1, -1), g1.reshape(1, -1), be1.reshape(1, -1)
    b2r, g2r, be2r = b2.reshape(1, -1), g2.reshape(1, -1), be2.reshape(1, -1)
    b3r = jnp.pad(b3, (0, DOUT_PAD - DOUT)).reshape(1, -1)

    degp = _sc_degree(dst_r2, ones16, zeros16)
    h1a, h1b = _tc_first(xp, W1, degp)
    p1a = _sc_aggregate(h1a, src_r, dst_r, zeros64)
    p1b = _sc_aggregate(h1b, src_r, dst_r, zeros64)
    h2a, h2b = _tc_mid(p1a, p1b, h1a, h1b, degp, b1r, g1r, be1r, W2, True)
    p2a = _sc_aggregate(h2a, src_r, dst_r, zeros64)
    p2b = _sc_aggregate(h2b, src_r, dst_r, zeros64)
    (h3,) = _tc_mid(p2a, p2b, h2a, h2b, degp, b2r, g2r, be2r, W3p, False)
    p3 = _sc_aggregate(h3, src_r, dst_r, zeros64)
    return _tc_final(p3, h3, degp, b3r)
